# baseline (speedup 1.0000x reference)
import functools

import jax
import jax.numpy as jnp
from jax import lax
from jax.experimental import pallas as pl
from jax.experimental.pallas import tpu as pltpu
from jax.experimental.pallas import tpu_sc as plsc

N_CLASSES = 1000
BATCH = 1024
SEQ = 20
D = N_CLASSES
DT = D // 8
DTA = 64
DTB = DT - DTA

NUM_CORES = 2
NUM_SUBCORES = 16
NW = NUM_CORES * NUM_SUBCORES
B_PER_W = BATCH // NW
TB_MINOR = B_PER_W + 1
NSTEP = 63

_mesh = plsc.VectorSubcoreMesh(core_axis_name="c", subcore_axis_name="s")


@functools.partial(
    pl.kernel,
    out_type=jax.ShapeDtypeStruct((SEQ, DT, 8, 8, 128), jnp.float32),
    mesh=_mesh,
    compiler_params=pltpu.CompilerParams(
        use_tc_tiling_on_sc=False, needs_layout_passes=False),
    scratch_types=[
        pltpu.VMEM((SEQ, B_PER_W), jnp.int32),
        pltpu.VMEM((B_PER_W, D), jnp.float32),
        pltpu.VMEM((B_PER_W, D), jnp.float32),
        pltpu.VMEM((DTA, 8, TB_MINOR), jnp.float32),
        pltpu.VMEM((DTB, 8, TB_MINOR), jnp.float32),
        pltpu.SemaphoreType.DMA,
        pltpu.SemaphoreType.DMA,
        pltpu.SemaphoreType.DMA,
        pltpu.SemaphoreType.DMA,
    ],
)
def _gather_t(idx_hbm, w_hbm, out_hbm, idx_t, gb0, gb1, tba, tbb,
              gs0, gs1, osa, osb):
    wid = lax.axis_index("s") * NUM_CORES + lax.axis_index("c")
    pltpu.sync_copy(idx_hbm.at[:, pl.ds(wid * B_PER_W, B_PER_W)], idx_t)
    bt = wid // 4
    off = (wid % 4) * B_PER_W

    iota16 = lax.iota(jnp.int32, 16)
    bconsts = [jnp.full((16,), b, jnp.int32) for b in range(B_PER_W)]

    def transpose_half(gb, tb, g_lo, g_hi, dt_base):
        @plsc.parallel_loop(g_lo, g_hi, unroll=4)
        def _step(g):
            d0 = lax.min(g * 16, D - 16)
            dvec = iota16 + d0
            dt_v = dvec // 8 - dt_base
            dr_v = dvec % 8
            for b0 in range(0, B_PER_W, 8):
                xs = [gb[b0 + i, pl.ds(d0, 16)] for i in range(8)]
                for i in range(8):
                    plsc.store_scatter(
                        tb, [dt_v, dr_v, bconsts[b0 + i]], xs[i])

    def g_start(s, gb, gsem):
        pltpu.async_copy(w_hbm.at[idx_t.at[s]], gb, gsem)

    def g_wait(gb, gsem):
        pltpu.make_async_copy(w_hbm.at[idx_t.at[0]], gb, gsem).wait()

    def out_slice(s, lo, n):
        return out_hbm.at[s, pl.ds(lo, n), bt, :, pl.ds(off, B_PER_W)]

    def unit(s, gb, first=False):
        if not first:
            pltpu.make_async_copy(
                tba.at[:, :, pl.ds(0, B_PER_W)], out_slice(0, 0, DTA), osa
            ).wait()
        transpose_half(gb, tba, 0, DTA * 8 // 16, 0)
        pltpu.async_copy(
            tba.at[:, :, pl.ds(0, B_PER_W)], out_slice(s, 0, DTA), osa)
        if not first:
            pltpu.make_async_copy(
                tbb.at[:, :, pl.ds(0, B_PER_W)], out_slice(0, DTA, DTB), osb
            ).wait()
        transpose_half(gb, tbb, DTA * 8 // 16, NSTEP, DTA)
        pltpu.async_copy(
            tbb.at[:, :, pl.ds(0, B_PER_W)], out_slice(s, DTA, DTB), osb)

    g_start(0, gb0, gs0)
    g_start(1, gb1, gs1)
    g_wait(gb0, gs0)
    unit(0, gb0, first=True)
    g_start(2, gb0, gs0)
    g_wait(gb1, gs1)
    unit(1, gb1)
    g_start(3, gb1, gs1)

    def body(k, carry):
        s0 = 2 * k
        g_wait(gb0, gs0)
        unit(s0, gb0)
        g_start(s0 + 2, gb0, gs0)
        g_wait(gb1, gs1)
        unit(s0 + 1, gb1)
        g_start(s0 + 3, gb1, gs1)
        return carry
    lax.fori_loop(1, SEQ // 2 - 1, body, 0)

    g_wait(gb0, gs0)
    unit(SEQ - 2, gb0)
    g_wait(gb1, gs1)
    unit(SEQ - 1, gb1)
    pltpu.make_async_copy(
        tba.at[:, :, pl.ds(0, B_PER_W)], out_slice(0, 0, DTA), osa).wait()
    pltpu.make_async_copy(
        tbb.at[:, :, pl.ds(0, B_PER_W)], out_slice(0, DTA, DTB), osb).wait()


def kernel(inp, hidden, W):
    out5 = _gather_t(inp.T.astype(jnp.int32), W)
    out = out5.transpose(2, 4, 0, 1, 3).reshape(BATCH, SEQ, D)
    return (out, hidden)


# speedup vs baseline: 1.0267x; 1.0267x over previous
import functools

import jax
import jax.numpy as jnp
from jax import lax
from jax.experimental import pallas as pl
from jax.experimental.pallas import tpu as pltpu
from jax.experimental.pallas import tpu_sc as plsc

N_CLASSES = 1000
BATCH = 1024
SEQ = 20
D = N_CLASSES
DT = D // 8
DTA = 64
DTB = DT - DTA

NUM_CORES = 2
NUM_SUBCORES = 16
NW = NUM_CORES * NUM_SUBCORES
B_PER_W = BATCH // NW
TB_MINOR = B_PER_W + 1
NSTEP = 63

_mesh = plsc.VectorSubcoreMesh(core_axis_name="c", subcore_axis_name="s")


@functools.partial(
    pl.kernel,
    out_type=jax.ShapeDtypeStruct((SEQ, DT, 8, 8, 128), jnp.float32),
    mesh=_mesh,
    compiler_params=pltpu.CompilerParams(
        use_tc_tiling_on_sc=False, needs_layout_passes=False),
    scratch_types=[
        pltpu.VMEM((SEQ, B_PER_W), jnp.int32),
        pltpu.VMEM((B_PER_W, D), jnp.float32),
        pltpu.VMEM((B_PER_W, D), jnp.float32),
        pltpu.VMEM((DTA, 8, TB_MINOR), jnp.float32),
        pltpu.VMEM((DTB, 8, TB_MINOR), jnp.float32),
        pltpu.SemaphoreType.DMA,
        pltpu.SemaphoreType.DMA,
        pltpu.SemaphoreType.DMA,
        pltpu.SemaphoreType.DMA,
    ],
)
def _gather_t(idx_hbm, w_hbm, out_hbm, idx_t, gb0, gb1, tba, tbb,
              gs0, gs1, osa, osb):
    wid = lax.axis_index("s") * NUM_CORES + lax.axis_index("c")
    pltpu.sync_copy(idx_hbm.at[:, pl.ds(wid * B_PER_W, B_PER_W)], idx_t)
    bt = wid // 4
    off = (wid % 4) * B_PER_W

    iota16 = lax.iota(jnp.int32, 16)
    bconsts = [jnp.full((16,), b, jnp.int32) for b in range(B_PER_W)]

    def transpose_half(gb, tb, g_lo, g_hi, dt_base, clamp=True):
        @plsc.parallel_loop(g_lo, g_hi, unroll=2)
        def _step(g):
            d0 = lax.min(g * 16, D - 16) if clamp else g * 16
            dvec = iota16 + d0
            dt_v = dvec // 8 - dt_base
            dr_v = dvec % 8
            for b0 in range(0, B_PER_W, 8):
                xs = [gb[b0 + i, pl.ds(d0, 16)] for i in range(8)]
                for i in range(8):
                    plsc.store_scatter(
                        tb, [dt_v, dr_v, bconsts[b0 + i]], xs[i])

    def g_start(s, gb, gsem):
        pltpu.async_copy(w_hbm.at[idx_t.at[s]], gb, gsem)

    def g_wait(gb, gsem):
        pltpu.make_async_copy(w_hbm.at[idx_t.at[0]], gb, gsem).wait()

    def out_slice(s, lo, n):
        return out_hbm.at[s, pl.ds(lo, n), bt, :, pl.ds(off, B_PER_W)]

    def unit(s, gb, first=False):
        if not first:
            pltpu.make_async_copy(
                tba.at[:, :, pl.ds(0, B_PER_W)], out_slice(0, 0, DTA), osa
            ).wait()
        transpose_half(gb, tba, 0, DTA * 8 // 16, 0, clamp=False)
        pltpu.async_copy(
            tba.at[:, :, pl.ds(0, B_PER_W)], out_slice(s, 0, DTA), osa)
        if not first:
            pltpu.make_async_copy(
                tbb.at[:, :, pl.ds(0, B_PER_W)], out_slice(0, DTA, DTB), osb
            ).wait()
        transpose_half(gb, tbb, DTA * 8 // 16, NSTEP, DTA)
        pltpu.async_copy(
            tbb.at[:, :, pl.ds(0, B_PER_W)], out_slice(s, DTA, DTB), osb)

    g_start(0, gb0, gs0)
    g_start(1, gb1, gs1)
    g_wait(gb0, gs0)
    unit(0, gb0, first=True)
    g_start(2, gb0, gs0)
    g_wait(gb1, gs1)
    unit(1, gb1)
    g_start(3, gb1, gs1)

    def body(k, carry):
        s0 = 2 * k
        g_wait(gb0, gs0)
        unit(s0, gb0)
        g_start(s0 + 2, gb0, gs0)
        g_wait(gb1, gs1)
        unit(s0 + 1, gb1)
        g_start(s0 + 3, gb1, gs1)
        return carry
    lax.fori_loop(1, SEQ // 2 - 1, body, 0)

    g_wait(gb0, gs0)
    unit(SEQ - 2, gb0)
    g_wait(gb1, gs1)
    unit(SEQ - 1, gb1)
    pltpu.make_async_copy(
        tba.at[:, :, pl.ds(0, B_PER_W)], out_slice(0, 0, DTA), osa).wait()
    pltpu.make_async_copy(
        tbb.at[:, :, pl.ds(0, B_PER_W)], out_slice(0, DTA, DTB), osb).wait()


def kernel(inp, hidden, W):
    out5 = _gather_t(inp.T.astype(jnp.int32), W)
    out = out5.transpose(2, 4, 0, 1, 3).reshape(BATCH, SEQ, D)
    return (out, hidden)


# speedup vs baseline: 1.1673x; 1.1369x over previous
import functools

import jax
import jax.numpy as jnp
from jax import lax
from jax.experimental import pallas as pl
from jax.experimental.pallas import tpu as pltpu
from jax.experimental.pallas import tpu_sc as plsc

N_CLASSES = 1000
BATCH = 1024
SEQ = 20
D = N_CLASSES
DT = D // 8
DTA = 64
DTB = DT - DTA

NUM_CORES = 2
NUM_SUBCORES = 16
NW = NUM_CORES * NUM_SUBCORES
B_PER_W = BATCH // NW
TB_MINOR = B_PER_W + 1
NSTEP = 63

_mesh = plsc.VectorSubcoreMesh(core_axis_name="c", subcore_axis_name="s")


@functools.partial(
    pl.kernel,
    out_type=jax.ShapeDtypeStruct((SEQ, DT, 8, 8, 128), jnp.float32),
    mesh=_mesh,
    compiler_params=pltpu.CompilerParams(
        use_tc_tiling_on_sc=False, needs_layout_passes=False),
    scratch_types=[
        pltpu.VMEM((SEQ, B_PER_W), jnp.int32),
        pltpu.VMEM((B_PER_W, D), jnp.float32),
        pltpu.VMEM((B_PER_W, D), jnp.float32),
        pltpu.VMEM((DTA, 8, TB_MINOR), jnp.float32),
        pltpu.VMEM((DTB, 8, TB_MINOR), jnp.float32),
        pltpu.SemaphoreType.DMA,
        pltpu.SemaphoreType.DMA,
        pltpu.SemaphoreType.DMA,
        pltpu.SemaphoreType.DMA,
    ],
)
def _gather_t(idx_hbm, w_hbm, out_hbm, idx_t, gb0, gb1, tba, tbb,
              gs0, gs1, osa, osb):
    wid = lax.axis_index("s") * NUM_CORES + lax.axis_index("c")
    pltpu.sync_copy(idx_hbm.at[:, pl.ds(wid * B_PER_W, B_PER_W)], idx_t)
    bt = wid // 4
    off = (wid % 4) * B_PER_W

    iota16 = lax.iota(jnp.int32, 16)
    bconsts = [jnp.full((16,), b, jnp.int32) for b in range(B_PER_W)]

    def transpose_half(gb, tb, g_lo, g_hi, dt_base):
        @plsc.parallel_loop(g_lo, g_hi, unroll=2)
        def _step(g):
            d0 = lax.min(g * 16, D - 16)
            dvec = iota16 + d0
            dt_v = dvec // 8 - dt_base
            dr_v = dvec % 8
            for b0 in range(0, B_PER_W, 8):
                xs = [gb[b0 + i, pl.ds(d0, 16)] for i in range(8)]
                for i in range(8):
                    plsc.store_scatter(
                        tb, [dt_v, dr_v, bconsts[b0 + i]], xs[i])

    def g_start(s, gb, gsem):
        pltpu.async_copy(w_hbm.at[idx_t.at[s]], gb, gsem)

    def g_wait(gb, gsem):
        pltpu.make_async_copy(w_hbm.at[idx_t.at[0]], gb, gsem).wait()

    def out_slice(s, lo, n):
        return out_hbm.at[s, pl.ds(lo, n), bt, :, pl.ds(off, B_PER_W)]

    def unit(s, gb, first=False):
        if not first:
            pltpu.make_async_copy(
                tba.at[:, :, pl.ds(0, B_PER_W)], out_slice(0, 0, DTA), osa
            ).wait()
        transpose_half(gb, tba, 0, DTA * 8 // 16, 0)
        pltpu.async_copy(
            tba.at[:, :, pl.ds(0, B_PER_W)], out_slice(s, 0, DTA), osa)
        if not first:
            pltpu.make_async_copy(
                tbb.at[:, :, pl.ds(0, B_PER_W)], out_slice(0, DTA, DTB), osb
            ).wait()
        transpose_half(gb, tbb, DTA * 8 // 16, NSTEP, DTA)
        pltpu.async_copy(
            tbb.at[:, :, pl.ds(0, B_PER_W)], out_slice(s, DTA, DTB), osb)

    g_start(0, gb0, gs0)
    g_start(1, gb1, gs1)
    g_wait(gb0, gs0)
    unit(0, gb0, first=True)
    g_start(2, gb0, gs0)
    g_wait(gb1, gs1)
    unit(1, gb1)
    g_start(3, gb1, gs1)

    def body(k, carry):
        s0 = 2 * k
        g_wait(gb0, gs0)
        unit(s0, gb0)
        g_start(s0 + 2, gb0, gs0)
        g_wait(gb1, gs1)
        unit(s0 + 1, gb1)
        g_start(s0 + 3, gb1, gs1)
        return carry
    lax.fori_loop(1, SEQ // 2 - 1, body, 0)

    g_wait(gb0, gs0)
    unit(SEQ - 2, gb0)
    g_wait(gb1, gs1)
    unit(SEQ - 1, gb1)
    pltpu.make_async_copy(
        tba.at[:, :, pl.ds(0, B_PER_W)], out_slice(0, 0, DTA), osa).wait()
    pltpu.make_async_copy(
        tbb.at[:, :, pl.ds(0, B_PER_W)], out_slice(0, DTA, DTB), osb).wait()


def kernel(inp, hidden, W):
    out5 = _gather_t(inp.T.astype(jnp.int32), W)
    out = out5.transpose(2, 4, 0, 1, 3).reshape(BATCH, SEQ, D)
    return (out, hidden)
